# baseline (speedup 1.0000x reference)
import functools

import jax
import jax.numpy as jnp
from jax import lax
from jax.experimental import pallas as pl
from jax.experimental.pallas import tpu as pltpu
from jax.experimental.pallas import tpu_sc as plsc

BATCH = 16384
EMBED = 64
NUM_LABELS = 16
NUM_WORKERS = 32
BPW = BATCH // NUM_WORKERS
CHUNK = 128
NCHUNK = BPW // CHUNK
LANES = 16

_mesh = plsc.VectorSubcoreMesh(core_axis_name="c", subcore_axis_name="s")


@functools.partial(
    pl.kernel,
    out_type=jax.ShapeDtypeStruct((BATCH,), jnp.float32),
    mesh=_mesh,
    compiler_params=pltpu.CompilerParams(needs_layout_passes=False,
                                         use_tc_tiling_on_sc=True),
    scratch_types=[
        pltpu.VMEM((NCHUNK, CHUNK), jnp.int32),
        pltpu.VMEM((NCHUNK, CHUNK), jnp.int32),
        pltpu.VMEM((1, BPW), jnp.int32),
        pltpu.VMEM((BPW, 2 * EMBED), jnp.float32),
        pltpu.VMEM((NUM_LABELS * EMBED,), jnp.float32),
        pltpu.VMEM((BPW,), jnp.float32),
        pltpu.SemaphoreType.DMA,
    ],
)
def _pmi_dot(dataT_hbm, wtab_hbm, ltab_hbm, out_hbm,
             idx_v, idx2_v, lbl_v, rows_v, ltab_v, out_v, sem):
    wid = lax.axis_index("s") * 2 + lax.axis_index("c")
    base = wid * BPW

    for c in range(NCHUNK):
        pltpu.sync_copy(
            dataT_hbm.at[pl.ds(0, 1), pl.ds(base + c * CHUNK, CHUNK)],
            idx_v.at[pl.ds(c, 1)])
    pltpu.sync_copy(dataT_hbm.at[pl.ds(1, 1), pl.ds(base, BPW)], lbl_v)
    pltpu.sync_copy(ltab_hbm, ltab_v)
    for c in range(NCHUNK):
        for g in range(CHUNK // LANES):
            idx2_v[c, pl.ds(g * LANES, LANES)] = (
                idx_v[c, pl.ds(g * LANES, LANES)] >> 1)
    copies = [
        pltpu.async_copy(
            wtab_hbm.at[idx2_v.at[c]],
            rows_v.at[pl.ds(c * CHUNK, CHUNK)],
            sem,
        )
        for c in range(NCHUNK)
    ]
    for cp in copies:
        cp.wait()

    iota = lax.iota(jnp.int32, LANES)

    def group(g, carry):
        gbase = pl.multiple_of(g * LANES, LANES)
        lbl_vec = lbl_v.at[0][pl.ds(gbase, LANES)]
        par_vec = idx_v.at[g // (CHUNK // LANES)][
            pl.ds((g % (CHUNK // LANES)) * LANES, LANES)] & 1
        acc = jnp.zeros((LANES,), jnp.float32)
        for j in range(LANES):
            row = rows_v.at[gbase + j]
            woff = par_vec[j] * EMBED
            off = lbl_vec[j] * EMBED
            p = row[pl.ds(woff, LANES)] * ltab_v[pl.ds(off, LANES)]
            for c in range(1, EMBED // LANES):
                p = p + (row[pl.ds(woff + c * LANES, LANES)]
                         * ltab_v[pl.ds(off + c * LANES, LANES)])
            acc = jnp.where(iota == j, jnp.sum(p), acc)
        out_v[pl.ds(gbase, LANES)] = acc
        return carry

    lax.fori_loop(0, BPW // LANES, group, 0)

    pltpu.sync_copy(out_v, out_hbm.at[pl.ds(base, BPW)])


def kernel(data, target, word_embedding, label_embedding):
    del target
    return _pmi_dot(data.astype(jnp.int32).T,
                    word_embedding.reshape(VOCAB_HALF, 2 * EMBED),
                    label_embedding.reshape(-1))


VOCAB_HALF = 500000
